# baseline (speedup 1.0000x reference)
import functools

import jax
import jax.numpy as jnp
from jax import lax
from jax.experimental import pallas as pl
from jax.experimental.pallas import tpu as pltpu
from jax.experimental.pallas import tpu_sc as plsc

_NUM_CORES = 2
_NUM_SUBCORES = 16


def kernel(x, pe_weight):
    batch = x.shape[0]
    max_len, d = pe_weight.shape
    nworkers = _NUM_CORES * _NUM_SUBCORES
    rows_per_w = max_len // nworkers
    chunk = 32
    nchunks = rows_per_w // chunk

    mesh = plsc.VectorSubcoreMesh(core_axis_name="c", subcore_axis_name="s")

    @functools.partial(
        pl.kernel,
        mesh=mesh,
        out_type=jax.ShapeDtypeStruct((batch, max_len, d), pe_weight.dtype),
        scratch_types=[
            pltpu.VMEM((2, chunk, d), pe_weight.dtype),
            pltpu.SemaphoreType.DMA((2,)),
            pltpu.SemaphoreType.DMA((2,)),
        ],
    )
    def sc_kernel(pe_hbm, out_hbm, buf, rsem, wsem):
        wid = lax.axis_index("s") * _NUM_CORES + lax.axis_index("c")
        base = wid * rows_per_w

        def read(c, slot):
            return pltpu.make_async_copy(
                pe_hbm.at[pl.ds(base + c * chunk, chunk), :],
                buf.at[slot],
                rsem.at[slot],
            )

        def write(c, slot, b):
            return pltpu.make_async_copy(
                buf.at[slot],
                out_hbm.at[b, pl.ds(base + c * chunk, chunk), :],
                wsem.at[slot],
            )

        read(0, 0).start()
        for c in range(nchunks):
            sl = c & 1
            if c + 1 < nchunks:
                if c >= 1:
                    for b in range(batch):
                        write(c - 1, 1 - sl, b).wait()
                read(c + 1, 1 - sl).start()
            read(c, sl).wait()
            for b in range(batch):
                write(c, sl, b).start()
        for c in (nchunks - 2, nchunks - 1):
            for b in range(batch):
                write(c, c & 1, b).wait()

    return sc_kernel(pe_weight)


# speedup vs baseline: 1.7669x; 1.7669x over previous
import jax
import jax.numpy as jnp
from jax.experimental import pallas as pl


def _bcast_kernel(in_ref, out_ref):
    out_ref[...] = jnp.broadcast_to(in_ref[...][None], out_ref.shape)


def kernel(x, pe_weight):
    batch = x.shape[0]
    max_len, d = pe_weight.shape
    rows = 1024
    return pl.pallas_call(
        _bcast_kernel,
        grid=(max_len // rows,),
        in_specs=[pl.BlockSpec((rows, d), lambda i: (i, 0))],
        out_specs=pl.BlockSpec((batch, rows, d), lambda i: (0, i, 0)),
        out_shape=jax.ShapeDtypeStruct((batch, max_len, d), pe_weight.dtype),
    )(pe_weight)
